# baseline (speedup 1.0000x reference)
import functools

import jax
import jax.numpy as jnp
from jax import lax
from jax.experimental import pallas as pl
from jax.experimental.pallas import tpu as pltpu
from jax.experimental.pallas import tpu_sc as plsc

N = 10000
D = 128
E_RAW = 320000
E_TOT = E_RAW + N
NPAD = 10240
ACCW = 144
NC = 2
NS = 16
NW = NC * NS
K = 64
NB = 3
NCH = NB * (-(-E_TOT // (NW * K * NB)))
NCH0 = 204
NCH1 = 2 * NCH - NCH0
EPAD = (NCH0 + NCH1) * NS * K
RPT = NPAD // NS
G = NPAD // D

_f32 = jnp.float32


def _sc_body(src_h, dst_h, asv_h, adv_h, h_h, c_h, acc_o, den_o,
             asv_v, adv_v, c_v,
             sidx0, sidx1, sidx2, didx0, didx1, didx2,
             exb0, exb1, exb2, rows0, rows1, rows2,
             acc_sh, den_sh,
             gsem0, gsem1, gsem2, ssem0, ssem1, ssem2,
             dsem0, dsem1, dsem2, isem0, isem1, isem2):
    sidx = (sidx0, sidx1, sidx2)
    didx = (didx0, didx1, didx2)
    exb = (exb0, exb1, exb2)
    rows = (rows0, rows1, rows2)
    gsem = (gsem0, gsem1, gsem2)
    ssem = (ssem0, ssem1, ssem2)
    dsem = (dsem0, dsem1, dsem2)
    isem = (isem0, isem1, isem2)
    cid = lax.axis_index("c")
    sid = lax.axis_index("s")
    nch = jnp.where(cid == 0, NCH0, NCH1)
    wbase = K * jnp.where(cid == 0, sid * NCH0, NS * NCH0 + sid * NCH1)

    pltpu.sync_copy(asv_h, asv_v)
    pltpu.sync_copy(adv_h, adv_v)
    pltpu.sync_copy(c_h, c_v)
    Cv = c_v[pl.ds(0, 16)]

    def zr(i, _):
        for j in range(D // 16):
            rows0[i, pl.ds(j * 16, 16)] = jnp.zeros((16,), _f32)
        return 0
    lax.fori_loop(0, K, zr, 0)
    def zb(g, _):
        exb0[pl.ds(g * 16, 16)] = jnp.zeros((16,), _f32)
        return 0
    lax.fori_loop(0, K // 16, zb, 0)
    for t in range(RPT // K):
        pltpu.sync_copy(rows0, acc_sh.at[pl.ds(sid * RPT + t * K, K)])
        pltpu.sync_copy(exb0, den_sh.at[pl.ds(sid * RPT + t * K, K)])
    plsc.subcore_barrier()

    def stage_idx(c, b):
        base = wbase + c * K
        pltpu.async_copy(src_h.at[pl.ds(base, K)], sidx[b], isem[b])
        pltpu.async_copy(dst_h.at[pl.ds(base, K)], didx[b], isem[b])

    def wait_idx(c, b):
        base = wbase + c * K
        pltpu.make_async_copy(src_h.at[pl.ds(base, K)], sidx[b],
                              isem[b]).wait()
        pltpu.make_async_copy(dst_h.at[pl.ds(base, K)], didx[b],
                              isem[b]).wait()

    def wait_scatter(b):
        pltpu.make_async_copy(rows[b], acc_sh.at[didx[b]], ssem[b]).wait()
        pltpu.make_async_copy(exb[b], den_sh.at[didx[b]], dsem[b]).wait()

    pltpu.sync_copy(src_h.at[pl.ds(wbase, K)], sidx[0])
    pltpu.sync_copy(dst_h.at[pl.ds(wbase, K)], didx[0])
    pltpu.async_copy(h_h.at[sidx[0]], rows[0], gsem[0])
    stage_idx(1, 1)

    def sub_iter(c, b):
        bn = (b + 1) % NB
        bp = (b + 2) % NB

        @pl.when(c + 1 < nch)
        def _():
            wait_idx(c + 1, bn)
            pltpu.async_copy(h_h.at[sidx[bn]], rows[bn], gsem[bn])

        pltpu.make_async_copy(h_h.at[sidx[b]], rows[b], gsem[b]).wait()

        @plsc.parallel_loop(0, K // 16, unroll=2)
        def _(g):
            si = sidx[b][pl.ds(g * 16, 16)]
            di = didx[b][pl.ds(g * 16, 16)]
            e = plsc.load_gather(asv_v, [si]) + plsc.load_gather(adv_v, [di])
            e = jnp.where(e > 0, e, 0.2 * e)
            ex = jnp.exp(e - Cv)
            exb[b][pl.ds(g * 16, 16)] = ex
            for l in range(16):
                i = g * 16 + l
                s = ex[l]
                for j in range(D // 16):
                    rows[b][i, pl.ds(j * 16, 16)] = (
                        rows[b][i, pl.ds(j * 16, 16)] * s)

        pltpu.async_copy(rows[b], acc_sh.at[didx[b]], ssem[b], add=True)
        pltpu.async_copy(exb[b], den_sh.at[didx[b]], dsem[b], add=True)

        @pl.when(c >= 1)
        def _():
            wait_scatter(bp)

        @pl.when(c + 2 < nch)
        def _():
            stage_idx(c + 2, bp)

    def outer(cc, _):
        for b in range(NB):
            sub_iter(NB * cc + b, b)
        return 0
    lax.fori_loop(0, nch // NB, outer, 0)
    wait_scatter(NB - 1)
    plsc.subcore_barrier()

    pltpu.sync_copy(acc_sh.at[pl.ds(sid * RPT, RPT)],
                    acc_o.at[cid, pl.ds(sid * RPT, RPT)])
    pltpu.sync_copy(den_sh.at[pl.ds(sid * RPT, RPT)],
                    den_o.at[cid, pl.ds(sid * RPT, RPT)])


_sc_edge = functools.partial(
    pl.kernel,
    out_type=(jax.ShapeDtypeStruct((NC, NPAD, D), _f32),
              jax.ShapeDtypeStruct((NC, NPAD), _f32)),
    mesh=plsc.VectorSubcoreMesh(core_axis_name="c", subcore_axis_name="s",
                                num_cores=NC, num_subcores=NS),
    compiler_params=pltpu.CompilerParams(needs_layout_passes=False),
    scratch_types=(
        [pltpu.VMEM((NPAD,), _f32),
         pltpu.VMEM((NPAD,), _f32),
         pltpu.VMEM((D,), _f32)]
        + [pltpu.VMEM((K,), jnp.int32)] * (2 * NB)
        + [pltpu.VMEM((K,), _f32)] * NB
        + [pltpu.VMEM((K, D), _f32)] * NB
        + [pltpu.VMEM_SHARED((NPAD, D), _f32),
           pltpu.VMEM_SHARED((NPAD,), _f32)]
        + [pltpu.SemaphoreType.DMA] * (4 * NB)
    ),
)(_sc_body)


def _nt_dot(v_row, h):
    return lax.dot_general(v_row, h, (((1,), (1,)), ((), ())),
                           preferred_element_type=_f32)


def _emit_attn(h, as_ref, ad_ref, asv_ref, adv_ref, c_ref, m_scr):
    asv = _nt_dot(as_ref[...], h)
    adv = _nt_dot(ad_ref[...], h)
    asv_ref[...] = asv[None]
    adv_ref[...] = adv[None]
    i = pl.program_id(0)
    ms = jnp.max(asv)
    md = jnp.max(adv)

    @pl.when(i == 0)
    def _():
        m_scr[0] = ms
        m_scr[1] = md

    @pl.when(i != 0)
    def _():
        m_scr[0] = jnp.maximum(m_scr[0], ms)
        m_scr[1] = jnp.maximum(m_scr[1], md)

    @pl.when(i == G - 1)
    def _():
        M = m_scr[0] + m_scr[1]
        C = jnp.where(M > 0, M, 0.2 * M)
        c_ref[...] = jnp.full((1, D), C, _f32)


def _tc_first_body(x_ref, w_ref, as_ref, ad_ref,
                   h_ref, asv_ref, adv_ref, c_ref, m_scr):
    h = jnp.dot(x_ref[...], w_ref[...], preferred_element_type=_f32)
    h_ref[...] = h
    _emit_attn(h, as_ref, ad_ref, asv_ref, adv_ref, c_ref, m_scr)


def _tc_mid_body(acc_ref, den_ref, b_ref, w_ref, as_ref, ad_ref,
                 h_ref, asv_ref, adv_ref, c_ref, m_scr):
    a = acc_ref[0] + acc_ref[1]
    d = den_ref[0] + den_ref[1]
    inv = 1.0 / (d + 1e-30)
    hin = jnp.maximum(a * inv + b_ref[...], 0.0)
    h = jnp.dot(hin, w_ref[...], preferred_element_type=_f32)
    h_ref[...] = h
    _emit_attn(h, as_ref, ad_ref, asv_ref, adv_ref, c_ref, m_scr)


def _tc_last_body(acc_ref, den_ref, b_ref, out_ref):
    a = acc_ref[0] + acc_ref[1]
    d = den_ref[0] + den_ref[1]
    inv = 1.0 / (d + 1e-30)
    out_ref[...] = a * inv + b_ref[...]


_rowspec = pl.BlockSpec((D, D), lambda i: (i, 0))
_wspec = pl.BlockSpec((D, D), lambda i: (0, 0))
_vspec = pl.BlockSpec((1, D), lambda i: (0, 0))
_sspec = pl.BlockSpec((1, 1, D), lambda i: (i, 0, 0))
_accspec = pl.BlockSpec((NC, D, D), lambda i: (0, i, 0))
_denspec = pl.BlockSpec((NC, D, 1), lambda i: (0, i, 0))

_attn_out_specs = [_rowspec, _sspec, _sspec,
                   pl.BlockSpec((1, D), lambda i: (0, 0))]
_attn_out_shape = [jax.ShapeDtypeStruct((NPAD, D), _f32),
                   jax.ShapeDtypeStruct((G, 1, D), _f32),
                   jax.ShapeDtypeStruct((G, 1, D), _f32),
                   jax.ShapeDtypeStruct((1, D), _f32)]

_tc_first = pl.pallas_call(
    _tc_first_body,
    grid=(G,),
    in_specs=[_rowspec, _wspec, _vspec, _vspec],
    out_specs=_attn_out_specs,
    out_shape=_attn_out_shape,
    scratch_shapes=[pltpu.SMEM((2,), _f32)],
)

_tc_mid = pl.pallas_call(
    _tc_mid_body,
    grid=(G,),
    in_specs=[_accspec, _denspec, _vspec, _wspec, _vspec, _vspec],
    out_specs=_attn_out_specs,
    out_shape=_attn_out_shape,
    scratch_shapes=[pltpu.SMEM((2,), _f32)],
)

_tc_last = pl.pallas_call(
    _tc_last_body,
    grid=(G,),
    in_specs=[_accspec, _denspec, _vspec],
    out_specs=_rowspec,
    out_shape=jax.ShapeDtypeStruct((NPAD, D), _f32),
)


@jax.jit
def kernel(x, edge_index, W0, as0, ad0, b0, W1, as1, ad1, b1,
           W2, as2, ad2, b2, W3, as3, ad3, b3, W4, as4, ad4, b4):
    params = [(W0, as0, ad0, b0), (W1, as1, ad1, b1), (W2, as2, ad2, b2),
              (W3, as3, ad3, b3), (W4, as4, ad4, b4)]

    xpad = jnp.zeros((NPAD, D), _f32).at[:N].set(x)
    loop = jnp.arange(N, dtype=jnp.int32)
    pad = jnp.full((EPAD - E_TOT,), N, dtype=jnp.int32)
    src = jnp.concatenate([edge_index[0], loop, pad])
    dst = jnp.concatenate([edge_index[1], loop, pad])

    W, a_s, a_d, _ = params[0]
    h, asv, adv, c = _tc_first(xpad, W, a_s.reshape(1, D), a_d.reshape(1, D))
    for i in range(1, 5):
        acc, den = _sc_edge(src, dst, asv.reshape(NPAD), adv.reshape(NPAD),
                            h, c.reshape(D))
        W, a_s, a_d, _ = params[i]
        b_prev = params[i - 1][3]
        h, asv, adv, c = _tc_mid(acc, den.reshape(NC, NPAD, 1),
                                 b_prev.reshape(1, D), W,
                                 a_s.reshape(1, D), a_d.reshape(1, D))
    acc, den = _sc_edge(src, dst, asv.reshape(NPAD), adv.reshape(NPAD),
                        h, c.reshape(D))
    out = _tc_last(acc, den.reshape(NC, NPAD, 1), params[4][3].reshape(1, D))
    return out[:N]


# speedup vs baseline: 1.0177x; 1.0177x over previous
import functools

import jax
import jax.numpy as jnp
from jax import lax
from jax.experimental import pallas as pl
from jax.experimental.pallas import tpu as pltpu
from jax.experimental.pallas import tpu_sc as plsc

N = 10000
D = 128
E_RAW = 320000
E_TOT = E_RAW + N
NPAD = 10240
ACCW = 144
NC = 2
NS = 16
NW = NC * NS
K = 64
NB = 3
NCH = NB * (-(-E_TOT // (NW * K * NB)))
NCH0 = 195
NCH1 = 2 * NCH - NCH0
EPAD = (NCH0 + NCH1) * NS * K
RPT = NPAD // NS
G = NPAD // D

_f32 = jnp.float32


def _sc_body(src_h, dst_h, asv_h, adv_h, h_h, c_h, acc_o, den_o,
             asv_v, adv_v, c_v,
             sidx0, sidx1, sidx2, didx0, didx1, didx2,
             exb0, exb1, exb2, rows0, rows1, rows2,
             acc_sh, den_sh,
             gsem0, gsem1, gsem2, ssem0, ssem1, ssem2,
             dsem0, dsem1, dsem2, isem0, isem1, isem2):
    sidx = (sidx0, sidx1, sidx2)
    didx = (didx0, didx1, didx2)
    exb = (exb0, exb1, exb2)
    rows = (rows0, rows1, rows2)
    gsem = (gsem0, gsem1, gsem2)
    ssem = (ssem0, ssem1, ssem2)
    dsem = (dsem0, dsem1, dsem2)
    isem = (isem0, isem1, isem2)
    cid = lax.axis_index("c")
    sid = lax.axis_index("s")
    nch = jnp.where(cid == 0, NCH0, NCH1)
    wbase = K * jnp.where(cid == 0, sid * NCH0, NS * NCH0 + sid * NCH1)

    pltpu.sync_copy(asv_h, asv_v)
    pltpu.sync_copy(adv_h, adv_v)
    pltpu.sync_copy(c_h, c_v)
    Cv = c_v[pl.ds(0, 16)]

    def zr(i, _):
        for j in range(D // 16):
            rows0[i, pl.ds(j * 16, 16)] = jnp.zeros((16,), _f32)
        return 0
    lax.fori_loop(0, K, zr, 0)
    def zb(g, _):
        exb0[pl.ds(g * 16, 16)] = jnp.zeros((16,), _f32)
        return 0
    lax.fori_loop(0, K // 16, zb, 0)
    for t in range(RPT // K):
        pltpu.async_copy(rows0, acc_sh.at[pl.ds(sid * RPT + t * K, K)],
                         ssem[0])
        pltpu.async_copy(exb0, den_sh.at[pl.ds(sid * RPT + t * K, K)],
                         dsem[0])
    for t in range(RPT // K):
        pltpu.make_async_copy(rows0, acc_sh.at[pl.ds(sid * RPT + t * K, K)],
                              ssem[0]).wait()
        pltpu.make_async_copy(exb0, den_sh.at[pl.ds(sid * RPT + t * K, K)],
                              dsem[0]).wait()
    plsc.subcore_barrier()

    def stage_idx(c, b):
        base = wbase + c * K
        pltpu.async_copy(src_h.at[pl.ds(base, K)], sidx[b], isem[b])
        pltpu.async_copy(dst_h.at[pl.ds(base, K)], didx[b], isem[b])

    def wait_idx(c, b):
        base = wbase + c * K
        pltpu.make_async_copy(src_h.at[pl.ds(base, K)], sidx[b],
                              isem[b]).wait()
        pltpu.make_async_copy(dst_h.at[pl.ds(base, K)], didx[b],
                              isem[b]).wait()

    def wait_scatter(b):
        pltpu.make_async_copy(rows[b], acc_sh.at[didx[b]], ssem[b]).wait()
        pltpu.make_async_copy(exb[b], den_sh.at[didx[b]], dsem[b]).wait()

    pltpu.sync_copy(src_h.at[pl.ds(wbase, K)], sidx[0])
    pltpu.sync_copy(dst_h.at[pl.ds(wbase, K)], didx[0])
    pltpu.async_copy(h_h.at[sidx[0]], rows[0], gsem[0])
    stage_idx(1, 1)

    def sub_iter(c, b):
        bn = (b + 1) % NB
        bp = (b + 2) % NB

        @pl.when(c + 1 < nch)
        def _():
            wait_idx(c + 1, bn)
            pltpu.async_copy(h_h.at[sidx[bn]], rows[bn], gsem[bn])

        pltpu.make_async_copy(h_h.at[sidx[b]], rows[b], gsem[b]).wait()

        @plsc.parallel_loop(0, K // 16, unroll=2)
        def _(g):
            si = sidx[b][pl.ds(g * 16, 16)]
            di = didx[b][pl.ds(g * 16, 16)]
            e = plsc.load_gather(asv_v, [si]) + plsc.load_gather(adv_v, [di])
            e = jnp.where(e > 0, e, 0.2 * e)
            ex = jnp.exp(e - Cv)
            exb[b][pl.ds(g * 16, 16)] = ex
            for l in range(16):
                i = g * 16 + l
                s = ex[l]
                for j in range(D // 16):
                    rows[b][i, pl.ds(j * 16, 16)] = (
                        rows[b][i, pl.ds(j * 16, 16)] * s)

        pltpu.async_copy(rows[b], acc_sh.at[didx[b]], ssem[b], add=True)
        pltpu.async_copy(exb[b], den_sh.at[didx[b]], dsem[b], add=True)

        @pl.when(c >= 1)
        def _():
            wait_scatter(bp)

        @pl.when(c + 2 < nch)
        def _():
            stage_idx(c + 2, bp)

    def outer(cc, _):
        for b in range(NB):
            sub_iter(NB * cc + b, b)
        return 0
    lax.fori_loop(0, nch // NB, outer, 0)
    wait_scatter(NB - 1)
    plsc.subcore_barrier()

    pltpu.async_copy(acc_sh.at[pl.ds(sid * RPT, RPT)],
                     acc_o.at[cid, pl.ds(sid * RPT, RPT)], ssem[0])
    pltpu.async_copy(den_sh.at[pl.ds(sid * RPT, RPT)],
                     den_o.at[cid, pl.ds(sid * RPT, RPT)], dsem[0])
    pltpu.make_async_copy(acc_sh.at[pl.ds(sid * RPT, RPT)],
                          acc_o.at[cid, pl.ds(sid * RPT, RPT)], ssem[0]).wait()
    pltpu.make_async_copy(den_sh.at[pl.ds(sid * RPT, RPT)],
                          den_o.at[cid, pl.ds(sid * RPT, RPT)], dsem[0]).wait()


_sc_edge = functools.partial(
    pl.kernel,
    out_type=(jax.ShapeDtypeStruct((NC, NPAD, D), _f32),
              jax.ShapeDtypeStruct((NC, NPAD), _f32)),
    mesh=plsc.VectorSubcoreMesh(core_axis_name="c", subcore_axis_name="s",
                                num_cores=NC, num_subcores=NS),
    compiler_params=pltpu.CompilerParams(needs_layout_passes=False),
    scratch_types=(
        [pltpu.VMEM((NPAD,), _f32),
         pltpu.VMEM((NPAD,), _f32),
         pltpu.VMEM((D,), _f32)]
        + [pltpu.VMEM((K,), jnp.int32)] * (2 * NB)
        + [pltpu.VMEM((K,), _f32)] * NB
        + [pltpu.VMEM((K, D), _f32)] * NB
        + [pltpu.VMEM_SHARED((NPAD, D), _f32),
           pltpu.VMEM_SHARED((NPAD,), _f32)]
        + [pltpu.SemaphoreType.DMA] * (4 * NB)
    ),
)(_sc_body)


def _nt_dot(v_row, h):
    return lax.dot_general(v_row, h, (((1,), (1,)), ((), ())),
                           preferred_element_type=_f32)


def _emit_attn(h, as_ref, ad_ref, asv_ref, adv_ref, c_ref, m_scr):
    asv = _nt_dot(as_ref[...], h)
    adv = _nt_dot(ad_ref[...], h)
    asv_ref[...] = asv[None]
    adv_ref[...] = adv[None]
    i = pl.program_id(0)
    ms = jnp.max(asv)
    md = jnp.max(adv)

    @pl.when(i == 0)
    def _():
        m_scr[0] = ms
        m_scr[1] = md

    @pl.when(i != 0)
    def _():
        m_scr[0] = jnp.maximum(m_scr[0], ms)
        m_scr[1] = jnp.maximum(m_scr[1], md)

    @pl.when(i == G - 1)
    def _():
        M = m_scr[0] + m_scr[1]
        C = jnp.where(M > 0, M, 0.2 * M)
        c_ref[...] = jnp.full((1, D), C, _f32)


def _tc_first_body(x_ref, w_ref, as_ref, ad_ref,
                   h_ref, asv_ref, adv_ref, c_ref, m_scr):
    h = jnp.dot(x_ref[...], w_ref[...], preferred_element_type=_f32)
    h_ref[...] = h
    _emit_attn(h, as_ref, ad_ref, asv_ref, adv_ref, c_ref, m_scr)


def _tc_mid_body(acc_ref, den_ref, b_ref, w_ref, as_ref, ad_ref,
                 h_ref, asv_ref, adv_ref, c_ref, m_scr):
    a = acc_ref[0] + acc_ref[1]
    d = den_ref[0] + den_ref[1]
    inv = 1.0 / (d + 1e-30)
    hin = jnp.maximum(a * inv + b_ref[...], 0.0)
    h = jnp.dot(hin, w_ref[...], preferred_element_type=_f32)
    h_ref[...] = h
    _emit_attn(h, as_ref, ad_ref, asv_ref, adv_ref, c_ref, m_scr)


def _tc_last_body(acc_ref, den_ref, b_ref, out_ref):
    a = acc_ref[0] + acc_ref[1]
    d = den_ref[0] + den_ref[1]
    inv = 1.0 / (d + 1e-30)
    out_ref[...] = a * inv + b_ref[...]


_rowspec = pl.BlockSpec((D, D), lambda i: (i, 0))
_wspec = pl.BlockSpec((D, D), lambda i: (0, 0))
_vspec = pl.BlockSpec((1, D), lambda i: (0, 0))
_sspec = pl.BlockSpec((1, 1, D), lambda i: (i, 0, 0))
_accspec = pl.BlockSpec((NC, D, D), lambda i: (0, i, 0))
_denspec = pl.BlockSpec((NC, D, 1), lambda i: (0, i, 0))

_attn_out_specs = [_rowspec, _sspec, _sspec,
                   pl.BlockSpec((1, D), lambda i: (0, 0))]
_attn_out_shape = [jax.ShapeDtypeStruct((NPAD, D), _f32),
                   jax.ShapeDtypeStruct((G, 1, D), _f32),
                   jax.ShapeDtypeStruct((G, 1, D), _f32),
                   jax.ShapeDtypeStruct((1, D), _f32)]

_tc_first = pl.pallas_call(
    _tc_first_body,
    grid=(G,),
    in_specs=[_rowspec, _wspec, _vspec, _vspec],
    out_specs=_attn_out_specs,
    out_shape=_attn_out_shape,
    scratch_shapes=[pltpu.SMEM((2,), _f32)],
)

_tc_mid = pl.pallas_call(
    _tc_mid_body,
    grid=(G,),
    in_specs=[_accspec, _denspec, _vspec, _wspec, _vspec, _vspec],
    out_specs=_attn_out_specs,
    out_shape=_attn_out_shape,
    scratch_shapes=[pltpu.SMEM((2,), _f32)],
)

_tc_last = pl.pallas_call(
    _tc_last_body,
    grid=(G,),
    in_specs=[_accspec, _denspec, _vspec],
    out_specs=_rowspec,
    out_shape=jax.ShapeDtypeStruct((NPAD, D), _f32),
)


@jax.jit
def kernel(x, edge_index, W0, as0, ad0, b0, W1, as1, ad1, b1,
           W2, as2, ad2, b2, W3, as3, ad3, b3, W4, as4, ad4, b4):
    params = [(W0, as0, ad0, b0), (W1, as1, ad1, b1), (W2, as2, ad2, b2),
              (W3, as3, ad3, b3), (W4, as4, ad4, b4)]

    xpad = jnp.zeros((NPAD, D), _f32).at[:N].set(x)
    loop = jnp.arange(N, dtype=jnp.int32)
    pad = jnp.full((EPAD - E_TOT,), N, dtype=jnp.int32)
    src = jnp.concatenate([edge_index[0], loop, pad])
    dst = jnp.concatenate([edge_index[1], loop, pad])

    W, a_s, a_d, _ = params[0]
    h, asv, adv, c = _tc_first(xpad, W, a_s.reshape(1, D), a_d.reshape(1, D))
    for i in range(1, 5):
        acc, den = _sc_edge(src, dst, asv.reshape(NPAD), adv.reshape(NPAD),
                            h, c.reshape(D))
        W, a_s, a_d, _ = params[i]
        b_prev = params[i - 1][3]
        h, asv, adv, c = _tc_mid(acc, den.reshape(NC, NPAD, 1),
                                 b_prev.reshape(1, D), W,
                                 a_s.reshape(1, D), a_d.reshape(1, D))
    acc, den = _sc_edge(src, dst, asv.reshape(NPAD), adv.reshape(NPAD),
                        h, c.reshape(D))
    out = _tc_last(acc, den.reshape(NC, NPAD, 1), params[4][3].reshape(1, D))
    return out[:N]
